# baseline (speedup 1.0000x reference)
import functools

import jax
import jax.numpy as jnp
from jax import lax
from jax.experimental import pallas as pl
from jax.experimental.pallas import tpu as pltpu
from jax.experimental.pallas import tpu_sc as plsc

_NC = 2
_NS = 16
_NW = _NC * _NS

_CHUNK = 512


@functools.lru_cache(maxsize=None)
def _make_gather(B, D):
    b_per_w = B // _NW
    n_chunks = b_per_w // _CHUNK
    mesh = plsc.VectorSubcoreMesh(core_axis_name="c", subcore_axis_name="s")

    @functools.partial(
        pl.kernel,
        out_type=jax.ShapeDtypeStruct((B, D), jnp.float32),
        mesh=mesh,
        scratch_types=[
            pltpu.VMEM((_CHUNK,), jnp.int32),
            pltpu.VMEM((_CHUNK, D), jnp.float32),
            pltpu.SemaphoreType.DMA,
        ],
        compiler_params=pltpu.CompilerParams(use_tc_tiling_on_sc=False),
    )
    def gather_kernel(idx_hbm, table_hbm, out_hbm, idx_v, rows_v, sem):
        wid = lax.axis_index("s") * _NC + lax.axis_index("c")
        base_w = wid * b_per_w

        def body(g, carry):
            base = base_w + g * _CHUNK
            pltpu.sync_copy(idx_hbm.at[pl.ds(base, _CHUNK)], idx_v)
            pltpu.async_copy(table_hbm.at[idx_v], rows_v, sem).wait()
            pltpu.sync_copy(rows_v, out_hbm.at[pl.ds(base, _CHUNK)])
            return carry

        lax.fori_loop(0, n_chunks, body, 0)

    return gather_kernel


def kernel(x, weight):
    shape = x.shape
    B = x.size
    D = weight.shape[1]
    flat_idx = jnp.reshape(x, (B,)).astype(jnp.int32)
    out = _make_gather(B, D)(flat_idx, weight)
    return jnp.reshape(out, shape + (D,))


# speedup vs baseline: 1.0420x; 1.0420x over previous
import functools

import jax
import jax.numpy as jnp
from jax import lax
from jax.experimental import pallas as pl
from jax.experimental.pallas import tpu as pltpu
from jax.experimental.pallas import tpu_sc as plsc

_NC = 2
_NS = 16
_NW = _NC * _NS

_CHUNK = 800


@functools.lru_cache(maxsize=None)
def _make_gather(B, D):
    b_per_w = B // _NW
    n_chunks = b_per_w // _CHUNK
    assert n_chunks * _CHUNK == b_per_w and n_chunks % 2 == 0 and n_chunks >= 6
    mesh = plsc.VectorSubcoreMesh(core_axis_name="c", subcore_axis_name="s")

    @functools.partial(
        pl.kernel,
        out_type=jax.ShapeDtypeStruct((B, D), jnp.float32),
        mesh=mesh,
        scratch_types=[
            pltpu.VMEM((_CHUNK,), jnp.int32),
            pltpu.VMEM((_CHUNK,), jnp.int32),
            pltpu.VMEM((_CHUNK, D), jnp.float32),
            pltpu.VMEM((_CHUNK, D), jnp.float32),
            pltpu.SemaphoreType.DMA,
            pltpu.SemaphoreType.DMA,
            pltpu.SemaphoreType.DMA,
            pltpu.SemaphoreType.DMA,
            pltpu.SemaphoreType.DMA,
            pltpu.SemaphoreType.DMA,
        ],
        compiler_params=pltpu.CompilerParams(use_tc_tiling_on_sc=False),
    )
    def gather_kernel(idx_hbm, table_hbm, out_hbm, idx_v0, idx_v1,
                      rows_v0, rows_v1, si0, si1, sg0, sg1, ss0, ss1):
        idx_v = (idx_v0, idx_v1)
        rows_v = (rows_v0, rows_v1)
        wid = lax.axis_index("s") * _NC + lax.axis_index("c")
        base_w = wid * b_per_w
        sem_i = (si0, si1)
        sem_g = (sg0, sg1)
        sem_s = (ss0, ss1)

        def load_idx(g, b):
            pltpu.async_copy(
                idx_hbm.at[pl.ds(base_w + g * _CHUNK, _CHUNK)],
                idx_v[b], sem_i[b])

        def gather(b):
            pltpu.async_copy(table_hbm.at[idx_v[b]], rows_v[b], sem_g[b])

        def store(g, b):
            pltpu.async_copy(
                rows_v[b],
                out_hbm.at[pl.ds(base_w + g * _CHUNK, _CHUNK)], sem_s[b])

        load_idx(0, 0)
        load_idx(1, 1)
        pltpu.make_async_copy(idx_hbm.at[pl.ds(0, _CHUNK)],
                              idx_v[0], sem_i[0]).wait()
        gather(0)
        pltpu.make_async_copy(idx_hbm.at[pl.ds(0, _CHUNK)],
                              idx_v[1], sem_i[1]).wait()
        gather(1)
        pltpu.make_async_copy(table_hbm.at[idx_v[0]],
                              rows_v[0], sem_g[0]).wait()
        store(0, 0)
        load_idx(2, 0)
        pltpu.make_async_copy(table_hbm.at[idx_v[1]],
                              rows_v[1], sem_g[1]).wait()
        store(1, 1)
        load_idx(3, 1)

        def outer(o, carry):
            for b in range(2):
                g = o * 2 + b
                pltpu.make_async_copy(
                    rows_v[b], out_hbm.at[pl.ds(0, _CHUNK)],
                    sem_s[b]).wait()
                pltpu.make_async_copy(
                    idx_hbm.at[pl.ds(0, _CHUNK)], idx_v[b],
                    sem_i[b]).wait()
                gather(b)
                pltpu.make_async_copy(
                    table_hbm.at[idx_v[b]], rows_v[b],
                    sem_g[b]).wait()
                store(g, b)
                load_idx(g + 2, b)
            return carry

        lax.fori_loop(1, n_chunks // 2 - 1, outer, 0)

        for b in range(2):
            g = n_chunks - 2 + b
            pltpu.make_async_copy(
                rows_v[b], out_hbm.at[pl.ds(0, _CHUNK)], sem_s[b]).wait()
            pltpu.make_async_copy(
                idx_hbm.at[pl.ds(0, _CHUNK)], idx_v[b], sem_i[b]).wait()
            gather(b)
        for b in range(2):
            g = n_chunks - 2 + b
            pltpu.make_async_copy(
                table_hbm.at[idx_v[b]], rows_v[b], sem_g[b]).wait()
            store(g, b)
        for b in range(2):
            pltpu.make_async_copy(
                rows_v[b], out_hbm.at[pl.ds(0, _CHUNK)], sem_s[b]).wait()

    return gather_kernel


def kernel(x, weight):
    shape = x.shape
    B = x.size
    D = weight.shape[1]
    flat_idx = jnp.reshape(x, (B,)).astype(jnp.int32)
    out = _make_gather(B, D)(flat_idx, weight)
    return jnp.reshape(out, shape + (D,))
